# baseline (speedup 1.0000x reference)
import jax
import jax.numpy as jnp
from jax.experimental import pallas as pl
from jax.experimental.pallas import tpu as pltpu

_W = 48
_R1 = 38 * _W + 41
_R2 = 36 * _W + 39
_R3 = 34 * _W + 37
_R37 = 42 * 32
_NB = 8
_M0 = 7056 - 170
_TA = (((0,), (0,)), ((), ()))
_SHIFTS0 = [ki * 84 + kj for ki in range(3) for kj in range(3)]


def _enc_kernel(x_ref, w0_ref, b0_ref, w1_ref, b1_ref, w2_ref, b2_ref,
                w3_ref, b3_ref, o_ref, h0a_ref, h0b_ref):
    x = x_ref[0]
    f32 = jnp.float32
    bf16 = jnp.bfloat16
    zpad = jnp.zeros((8, _M0), bf16)
    xcat = jnp.concatenate(
        [p for s in _SHIFTS0 for p in (x[:, s:s + _M0], zpad)], axis=0)
    y = jax.lax.dot_general(xcat, w0_ref[...], _TA,
                            preferred_element_type=f32)
    h0a_ref[pl.ds(0, 6816)] = y[:6816, :128]
    h0b_ref[pl.ds(0, 6816)] = y[:6816, 128:]
    h = jnp.concatenate(
        [jnp.concatenate([r[pl.ds(2 * a * 84, _W, 2)] for a in range(41)],
                         axis=0)
         for r in (h0a_ref, h0b_ref)], axis=1)
    h = jnp.maximum(h + b0_ref[...], 0.0)
    for w_ref, b_ref, r_out in ((w1_ref, b1_ref, _R1),
                                (w2_ref, b2_ref, _R2),
                                (w3_ref, b3_ref, _R3)):
        rc = r_out + 2 * _W
        hcat = jnp.concatenate([h[kj: kj + rc] for kj in range(3)],
                               axis=1).astype(bf16)
        acc = None
        for ki in range(3):
            d = jnp.dot(hcat, w_ref[768 * ki: 768 * (ki + 1)],
                        preferred_element_type=f32)
            dd = d[ki * _W: ki * _W + r_out]
            acc = dd if acc is None else acc + dd
        h = jnp.maximum(acc + b_ref[...], 0.0)
    w37 = jnp.concatenate(
        [h[oh * _W: oh * _W + 37] for oh in range(35)] + [h[:_R37 - 35 * 37]],
        axis=0).astype(bf16)
    for g in range(_NB):
        o_ref[0, g] = w37[:, 32 * g: 32 * (g + 1)]


def _fc_ln_kernel(h_ref, w_ref, b_ref, g_ref, beta_ref, o_ref):
    y = jnp.dot(h_ref[...], w_ref[...], preferred_element_type=jnp.float32)
    y = y + b_ref[...]
    mean = jnp.mean(y, axis=-1, keepdims=True)
    var = jnp.mean((y - mean) ** 2, axis=-1, keepdims=True)
    out = (y - mean) * jax.lax.rsqrt(var + 1e-5) * g_ref[...] + beta_ref[...]
    o_ref[...] = jnp.tanh(out)


def _blockdiag(w):
    return jnp.kron(jnp.eye(_NB, dtype=w.dtype), w)


def kernel(obs, conv_w_0, conv_b_0, conv_w_1, conv_b_1, conv_w_2, conv_b_2,
           conv_w_3, conv_b_3, fc_w, fc_b, ln_gamma, ln_beta):
    n = obs.shape[0]
    nblk = n // _NB
    bf16 = jnp.bfloat16
    obs = obs.astype(jnp.float32)
    scale = jnp.where(jnp.max(obs) > 1.0, 1.0 / 255.0, 1.0)

    x = obs.astype(bf16).reshape(nblk, _NB * 9, 84 * 84)

    w0t = jnp.transpose(conv_w_0, (2, 3, 1, 0))
    pad8 = jnp.zeros((8, _NB * 32), jnp.float32)
    w0cat = jnp.concatenate(
        [p for ki in range(3) for kj in range(3)
         for p in (_blockdiag(w0t[ki, kj] * scale), pad8)],
        axis=0).astype(bf16)

    def wcat(w):
        t = jnp.transpose(w, (2, 3, 1, 0)).reshape(9, 32, 32)
        return jnp.concatenate([_blockdiag(t[k]) for k in range(9)],
                               axis=0).astype(bf16)

    wbig = [wcat(conv_w_1), wcat(conv_w_2), wcat(conv_w_3)]
    bbig = [jnp.tile(b, _NB).reshape(1, _NB * 32)
            for b in (conv_b_0, conv_b_1, conv_b_2, conv_b_3)]

    conv_flops = 2 * n * (_M0 * 9 * 32 * 9 + (_R1 + _R2 + _R3) * 9 * 32 * 32)
    h = pl.pallas_call(
        _enc_kernel,
        out_shape=jax.ShapeDtypeStruct((nblk, _NB, _R37, 32), bf16),
        grid=(nblk,),
        in_specs=[
            pl.BlockSpec((1, _NB * 9, 84 * 84), lambda i: (i, 0, 0)),
            pl.BlockSpec((720, _NB * 32), lambda i: (0, 0)),
            pl.BlockSpec((1, _NB * 32), lambda i: (0, 0)),
            pl.BlockSpec((9 * _NB * 32, _NB * 32), lambda i: (0, 0)),
            pl.BlockSpec((1, _NB * 32), lambda i: (0, 0)),
            pl.BlockSpec((9 * _NB * 32, _NB * 32), lambda i: (0, 0)),
            pl.BlockSpec((1, _NB * 32), lambda i: (0, 0)),
            pl.BlockSpec((9 * _NB * 32, _NB * 32), lambda i: (0, 0)),
            pl.BlockSpec((1, _NB * 32), lambda i: (0, 0)),
        ],
        out_specs=pl.BlockSpec((1, _NB, _R37, 32), lambda i: (i, 0, 0, 0)),
        scratch_shapes=[pltpu.VMEM((6888, 128), jnp.float32),
                        pltpu.VMEM((6888, 128), jnp.float32)],
        compiler_params=pltpu.CompilerParams(
            dimension_semantics=("parallel",),
            vmem_limit_bytes=60 * 1024 * 1024,
        ),
        cost_estimate=pl.CostEstimate(
            flops=conv_flops,
            transcendentals=0,
            bytes_accessed=2 * (nblk * _NB * 9 * 84 * 84
                                + nblk * _NB * _R37 * 32),
        ),
    )(x, w0cat, bbig[0], wbig[0], bbig[1], wbig[1], bbig[2], wbig[2], bbig[3])

    k_dim = _R37 * 32
    hflat = h.reshape(n, k_dim)
    bm = 32 if n % 32 == 0 else n
    out = pl.pallas_call(
        _fc_ln_kernel,
        out_shape=jax.ShapeDtypeStruct((n, 50), jnp.float32),
        grid=(n // bm,),
        in_specs=[
            pl.BlockSpec((bm, k_dim), lambda i: (i, 0)),
            pl.BlockSpec((k_dim, 50), lambda i: (0, 0)),
            pl.BlockSpec((1, 50), lambda i: (0, 0)),
            pl.BlockSpec((1, 50), lambda i: (0, 0)),
            pl.BlockSpec((1, 50), lambda i: (0, 0)),
        ],
        out_specs=pl.BlockSpec((bm, 50), lambda i: (i, 0)),
        compiler_params=pltpu.CompilerParams(
            dimension_semantics=("parallel",),
            vmem_limit_bytes=60 * 1024 * 1024,
        ),
        cost_estimate=pl.CostEstimate(
            flops=2 * n * k_dim * 50,
            transcendentals=n * 50,
            bytes_accessed=2 * (n * k_dim + k_dim * 50) + 4 * n * 50,
        ),
    )(hflat, fc_w.astype(bf16),
      fc_b.reshape(1, 50), ln_gamma.reshape(1, 50), ln_beta.reshape(1, 50))
    return out


# speedup vs baseline: 1.0411x; 1.0411x over previous
import jax
import jax.numpy as jnp
from jax.experimental import pallas as pl
from jax.experimental.pallas import tpu as pltpu

_W = 48
_R1 = 38 * _W + 41
_R2 = 36 * _W + 39
_R3 = 34 * _W + 37
_R37 = 42 * 32
_NB = 8
_M0 = 7056 - 170
_TA = (((0,), (0,)), ((), ()))
_SHIFTS0 = [ki * 84 + kj for ki in range(3) for kj in range(3)]


def _enc_kernel(x_ref, w0_ref, b0_ref, w1_ref, b1_ref, w2_ref, b2_ref,
                w3_ref, b3_ref, o_ref, h0a_ref, h0b_ref):
    x = x_ref[0]
    f32 = jnp.float32
    bf16 = jnp.bfloat16
    zpad = jnp.zeros((8, _M0), bf16)
    xcat = jnp.concatenate(
        [p for s in _SHIFTS0 for p in (x[:, s:s + _M0], zpad)], axis=0)
    y = jax.lax.dot_general(xcat, w0_ref[...], _TA,
                            preferred_element_type=f32)
    h0 = jnp.maximum(y + b0_ref[...], 0.0)
    h0a_ref[pl.ds(0, _M0)] = h0[:, :128]
    h0b_ref[pl.ds(0, _M0)] = h0[:, 128:]
    h = jnp.concatenate(
        [jnp.concatenate([r[pl.ds(2 * a * 84, _W, 2)] for a in range(41)],
                         axis=0)
         for r in (h0a_ref, h0b_ref)], axis=1).astype(bf16)
    for w_ref, b_ref, r_out in ((w1_ref, b1_ref, _R1),
                                (w2_ref, b2_ref, _R2),
                                (w3_ref, b3_ref, _R3)):
        hs = (h, h[1:], h[2:])
        hcat = jnp.concatenate(
            [hs[kj][ki * _W: ki * _W + r_out]
             for ki in range(3) for kj in range(3)], axis=1)
        y = jnp.dot(hcat, w_ref[...], preferred_element_type=f32)
        h = jnp.maximum(y + b_ref[...], 0.0).astype(bf16)
    w37 = jnp.concatenate(
        [h[oh * _W: oh * _W + 37] for oh in range(35)] + [h[:_R37 - 35 * 37]],
        axis=0)
    for g in range(_NB):
        o_ref[0, g] = w37[:, 32 * g: 32 * (g + 1)]


def _fc_ln_kernel(h_ref, w_ref, b_ref, g_ref, beta_ref, o_ref):
    y = jnp.dot(h_ref[...], w_ref[...], preferred_element_type=jnp.float32)
    y = y + b_ref[...]
    mean = jnp.mean(y, axis=-1, keepdims=True)
    var = jnp.mean((y - mean) ** 2, axis=-1, keepdims=True)
    out = (y - mean) * jax.lax.rsqrt(var + 1e-5) * g_ref[...] + beta_ref[...]
    o_ref[...] = jnp.tanh(out)


def _blockdiag(w):
    return jnp.kron(jnp.eye(_NB, dtype=w.dtype), w)


def kernel(obs, conv_w_0, conv_b_0, conv_w_1, conv_b_1, conv_w_2, conv_b_2,
           conv_w_3, conv_b_3, fc_w, fc_b, ln_gamma, ln_beta):
    n = obs.shape[0]
    nblk = n // _NB
    bf16 = jnp.bfloat16
    obs = obs.astype(jnp.float32)
    scale = jnp.where(jnp.max(obs) > 1.0, 1.0 / 255.0, 1.0)

    x = obs.astype(bf16).reshape(nblk, _NB * 9, 84 * 84)

    w0t = jnp.transpose(conv_w_0, (2, 3, 1, 0))
    pad8 = jnp.zeros((8, _NB * 32), jnp.float32)
    w0cat = jnp.concatenate(
        [p for ki in range(3) for kj in range(3)
         for p in (_blockdiag(w0t[ki, kj] * scale), pad8)],
        axis=0).astype(bf16)

    def wcat(w):
        t = jnp.transpose(w, (2, 3, 1, 0)).reshape(9, 32, 32)
        return jnp.concatenate([_blockdiag(t[k]) for k in range(9)],
                               axis=0).astype(bf16)

    wbig = [wcat(conv_w_1), wcat(conv_w_2), wcat(conv_w_3)]
    bbig = [jnp.tile(b, _NB).reshape(1, _NB * 32)
            for b in (conv_b_0, conv_b_1, conv_b_2, conv_b_3)]

    conv_flops = 2 * n * (_M0 * 9 * 32 * 9 + (_R1 + _R2 + _R3) * 9 * 32 * 32)
    h = pl.pallas_call(
        _enc_kernel,
        out_shape=jax.ShapeDtypeStruct((nblk, _NB, _R37, 32), bf16),
        grid=(nblk,),
        in_specs=[
            pl.BlockSpec((1, _NB * 9, 84 * 84), lambda i: (i, 0, 0)),
            pl.BlockSpec((720, _NB * 32), lambda i: (0, 0)),
            pl.BlockSpec((1, _NB * 32), lambda i: (0, 0)),
            pl.BlockSpec((9 * _NB * 32, _NB * 32), lambda i: (0, 0)),
            pl.BlockSpec((1, _NB * 32), lambda i: (0, 0)),
            pl.BlockSpec((9 * _NB * 32, _NB * 32), lambda i: (0, 0)),
            pl.BlockSpec((1, _NB * 32), lambda i: (0, 0)),
            pl.BlockSpec((9 * _NB * 32, _NB * 32), lambda i: (0, 0)),
            pl.BlockSpec((1, _NB * 32), lambda i: (0, 0)),
        ],
        out_specs=pl.BlockSpec((1, _NB, _R37, 32), lambda i: (i, 0, 0, 0)),
        scratch_shapes=[pltpu.VMEM((6888, 128), jnp.float32),
                        pltpu.VMEM((6888, 128), jnp.float32)],
        compiler_params=pltpu.CompilerParams(
            dimension_semantics=("parallel",),
            vmem_limit_bytes=60 * 1024 * 1024,
        ),
        cost_estimate=pl.CostEstimate(
            flops=conv_flops,
            transcendentals=0,
            bytes_accessed=2 * (nblk * _NB * 9 * 84 * 84
                                + nblk * _NB * _R37 * 32),
        ),
    )(x, w0cat, bbig[0], wbig[0], bbig[1], wbig[1], bbig[2], wbig[2], bbig[3])

    k_dim = _R37 * 32
    hflat = h.reshape(n, k_dim)
    bm = 32 if n % 32 == 0 else n
    out = pl.pallas_call(
        _fc_ln_kernel,
        out_shape=jax.ShapeDtypeStruct((n, 50), jnp.float32),
        grid=(n // bm,),
        in_specs=[
            pl.BlockSpec((bm, k_dim), lambda i: (i, 0)),
            pl.BlockSpec((k_dim, 50), lambda i: (0, 0)),
            pl.BlockSpec((1, 50), lambda i: (0, 0)),
            pl.BlockSpec((1, 50), lambda i: (0, 0)),
            pl.BlockSpec((1, 50), lambda i: (0, 0)),
        ],
        out_specs=pl.BlockSpec((bm, 50), lambda i: (i, 0)),
        compiler_params=pltpu.CompilerParams(
            dimension_semantics=("parallel",),
            vmem_limit_bytes=60 * 1024 * 1024,
        ),
        cost_estimate=pl.CostEstimate(
            flops=2 * n * k_dim * 50,
            transcendentals=n * 50,
            bytes_accessed=2 * (n * k_dim + k_dim * 50) + 4 * n * 50,
        ),
    )(hflat, fc_w.astype(bf16),
      fc_b.reshape(1, 50), ln_gamma.reshape(1, 50), ln_beta.reshape(1, 50))
    return out
